# Initial kernel scaffold; baseline (speedup 1.0000x reference)
import jax
import jax.numpy as jnp
from jax.experimental import pallas as pl


def kernel(x, edge_index, W1, a_src1, a_dst1, b1, W2, a_src2, a_dst2, b2):
    raise NotImplementedError("write your pallas kernel here")


# speedup vs baseline: 28.3742x; 28.3742x over previous
import functools

import jax
import jax.numpy as jnp
from jax import lax
from jax.experimental import pallas as pl
from jax.experimental.pallas import tpu as pltpu
from jax.experimental.pallas import tpu_sc as plsc

F32 = jnp.float32
I32 = jnp.int32

NCORES = 2
NSUB = 16
LANES = 16

N = 10000
D_IN = 128
HEADS = 4
CH = 64
HC = HEADS * CH
E = 320000

W1AUG = 144
W2AUG = 80
K = 80
NT = N // NSUB
EPC = 64
EPC_LAST = NT - 9 * EPC

BN = 1000
GRID = N // BN


def _tc_a_body(x_ref, w1_ref, ab_ref, hp_ref, al_ref):
    h = jnp.dot(x_ref[...], w1_ref[...], preferred_element_type=F32)
    ab = jnp.dot(h, ab_ref[...], preferred_element_type=F32)
    ones2 = jnp.ones((BN, 2), F32)
    pad14 = jnp.zeros((BN, 14), F32)
    hp_ref[0] = jnp.concatenate([h[:, 0:128], ones2, pad14], axis=1)
    hp_ref[1] = jnp.concatenate([h[:, 128:256], ones2, pad14], axis=1)
    padal = jnp.zeros((BN, 12), F32)
    al_ref[0] = jnp.concatenate([ab[:, 0:2], ab[:, 4:6], padal], axis=1)
    al_ref[1] = jnp.concatenate([ab[:, 2:4], ab[:, 6:8], padal], axis=1)


def _tc_a(x, W1, Aboth):
    return pl.pallas_call(
        _tc_a_body,
        grid=(GRID,),
        in_specs=[
            pl.BlockSpec((BN, D_IN), lambda i: (i, 0)),
            pl.BlockSpec((D_IN, HC), lambda i: (0, 0)),
            pl.BlockSpec((HC, 2 * HEADS), lambda i: (0, 0)),
        ],
        out_specs=[
            pl.BlockSpec((2, BN, W1AUG), lambda i: (0, i, 0)),
            pl.BlockSpec((2, BN, 16), lambda i: (0, i, 0)),
        ],
        out_shape=[
            jax.ShapeDtypeStruct((2, N, W1AUG), F32),
            jax.ShapeDtypeStruct((2, N, 16), F32),
        ],
    )(x, W1, Aboth)


def _sc1_body(hp_hbm, al_hbm, src_hbm, dst_hbm, b1_hbm, h1_hbm,
              btab, sidx, gidx, gdidx, didx, rows, asrcr, adstr,
              w0b, w1b, r0b, r1b, hrb, outb, acc, gsem, asem, bsem):
    c = lax.axis_index("c")
    s = lax.axis_index("s")
    iota = lax.iota(I32, LANES)
    z16 = jnp.zeros((LANES,), I32)
    zf16 = jnp.zeros((LANES,), F32)

    def _zrow(i, _):
        for j in range(W1AUG // LANES):
            rows[i, pl.ds(j * LANES, LANES)] = zf16
        return 0
    lax.fori_loop(0, EPC, _zrow, 0)
    for r in range(9):
        pltpu.sync_copy(rows.at[pl.ds(0, EPC)],
                        acc.at[pl.ds(s * NT + r * EPC, EPC)])
    pltpu.sync_copy(rows.at[pl.ds(0, EPC_LAST)],
                    acc.at[pl.ds(s * NT + 9 * EPC, EPC_LAST)])

    pltpu.sync_copy(b1_hbm.at[pl.ds(c * 128, 128)], btab)
    plsc.subcore_barrier()

    ec = E // NSUB
    nchunks = ec // K
    coff = c * N

    def _chunk(g, _):
        base = s * ec + g * K
        pltpu.sync_copy(src_hbm.at[pl.ds(base, K)], sidx)
        pltpu.sync_copy(dst_hbm.at[pl.ds(base, K)], didx)
        for j in range(K // LANES):
            sl = pl.ds(j * LANES, LANES)
            gidx[sl] = sidx[sl] + coff
            gdidx[sl] = didx[sl] + coff
        d1 = pltpu.async_copy(hp_hbm.at[gidx], rows, gsem)
        d2 = pltpu.async_copy(al_hbm.at[gidx], asrcr, asem)
        d3 = pltpu.async_copy(al_hbm.at[gdidx], adstr, bsem)
        d2.wait()
        d3.wait()
        for j in range(K // LANES):
            sl = pl.ds(j * LANES, LANES)
            l16 = iota + j * LANES
            e0 = (plsc.load_gather(asrcr, [l16, z16])
                  + plsc.load_gather(adstr, [l16, z16 + 2]))
            e1 = (plsc.load_gather(asrcr, [l16, z16 + 1])
                  + plsc.load_gather(adstr, [l16, z16 + 3]))
            w0b[sl] = jnp.exp(jnp.maximum(e0, 0.2 * e0))
            w1b[sl] = jnp.exp(jnp.maximum(e1, 0.2 * e1))
        d1.wait()

        def _edge(e, _):
            w0 = w0b[pl.ds(e, LANES)][0]
            w1 = w1b[pl.ds(e, LANES)][0]
            for v in range(4):
                sl = pl.ds(v * LANES, LANES)
                rows[e, sl] = rows[e, sl] * w0
            for v in range(4, 8):
                sl = pl.ds(v * LANES, LANES)
                rows[e, sl] = rows[e, sl] * w1
            wsel = jnp.where(iota == 0, w0, w1)
            sl = pl.ds(128, LANES)
            rows[e, sl] = rows[e, sl] * wsel
            return 0
        lax.fori_loop(0, K, _edge, 0)
        pltpu.sync_copy(rows, acc.at[didx], add=True)
        return 0
    lax.fori_loop(0, nchunks, _chunk, 0)
    plsc.subcore_barrier()

    for ch in range(10):
        size = EPC if ch < 9 else EPC_LAST
        lo = s * NT + ch * EPC
        pltpu.sync_copy(acc.at[pl.ds(lo, size)], rows.at[pl.ds(0, size)])
        pltpu.sync_copy(hp_hbm.at[pl.ds(coff + lo, size)],
                        hrb.at[pl.ds(0, size)])
        pltpu.sync_copy(al_hbm.at[pl.ds(coff + lo, size)],
                        asrcr.at[pl.ds(0, size)])
        for grp in range(4):
            l16 = grp * LANES + iota
            e0 = (plsc.load_gather(asrcr, [l16, z16])
                  + plsc.load_gather(asrcr, [l16, z16 + 2]))
            e1 = (plsc.load_gather(asrcr, [l16, z16 + 1])
                  + plsc.load_gather(asrcr, [l16, z16 + 3]))
            w0 = jnp.exp(jnp.maximum(e0, 0.2 * e0))
            w1 = jnp.exp(jnp.maximum(e1, 0.2 * e1))
            den0 = plsc.load_gather(rows, [l16, z16 + 128]) + w0 + 1e-16
            den1 = plsc.load_gather(rows, [l16, z16 + 129]) + w1 + 1e-16
            sl = pl.ds(grp * LANES, LANES)
            w0b[sl] = w0
            w1b[sl] = w1
            r0b[sl] = 1.0 / den0
            r1b[sl] = 1.0 / den1

        def _node(i, _):
            w0 = w0b[pl.ds(i, LANES)][0]
            w1 = w1b[pl.ds(i, LANES)][0]
            r0 = r0b[pl.ds(i, LANES)][0]
            r1 = r1b[pl.ds(i, LANES)][0]
            for v in range(8):
                sl = pl.ds(v * LANES, LANES)
                w = w0 if v < 4 else w1
                r = r0 if v < 4 else r1
                val = (rows[i, sl] + w * hrb[i, sl]) * r + btab[sl]
                outb[i, sl] = jnp.maximum(val, 0.0)
            return 0
        lax.fori_loop(0, size, _node, 0)
        pltpu.sync_copy(outb.at[pl.ds(0, size)],
                        h1_hbm.at[pl.ds(coff + lo, size)])


def _sc1(hpflat, alflat, srcv, dstv, b1p):
    mesh = plsc.VectorSubcoreMesh(core_axis_name="c", subcore_axis_name="s",
                                  num_cores=NCORES, num_subcores=NSUB)
    f = functools.partial(
        pl.kernel,
        out_type=jax.ShapeDtypeStruct((2 * N, 128), F32),
        mesh=mesh,
        scratch_types=[
            pltpu.VMEM((128,), F32),
            pltpu.VMEM((K,), I32),
            pltpu.VMEM((K,), I32),
            pltpu.VMEM((K,), I32),
            pltpu.VMEM((K,), I32),
            pltpu.VMEM((K, W1AUG), F32),
            pltpu.VMEM((K, 16), F32),
            pltpu.VMEM((K, 16), F32),
            pltpu.VMEM((K + LANES,), F32),
            pltpu.VMEM((K + LANES,), F32),
            pltpu.VMEM((K + LANES,), F32),
            pltpu.VMEM((K + LANES,), F32),
            pltpu.VMEM((EPC, W1AUG), F32),
            pltpu.VMEM((EPC, 128), F32),
            pltpu.VMEM_SHARED((N, W1AUG), F32),
            pltpu.SemaphoreType.DMA,
            pltpu.SemaphoreType.DMA,
            pltpu.SemaphoreType.DMA,
        ],
        compiler_params=pltpu.CompilerParams(use_tc_tiling_on_sc=False, needs_layout_passes=False),
    )(_sc1_body)
    return f(hpflat, alflat, srcv, dstv, b1p)


def _tc_b_body(h1_ref, w2_ref, a2_ref, hp2_ref, al2_ref):
    h2 = (jnp.dot(h1_ref[0], w2_ref[0], preferred_element_type=F32)
          + jnp.dot(h1_ref[1], w2_ref[1], preferred_element_type=F32))
    al2 = jnp.dot(h2, a2_ref[...].T, preferred_element_type=F32)
    hp2_ref[...] = jnp.concatenate(
        [h2, jnp.ones((BN, 1), F32), jnp.zeros((BN, 15), F32)], axis=1)
    al2_ref[...] = jnp.concatenate([al2, jnp.zeros((BN, 14), F32)], axis=1)


def _tc_b(h1, W2p, a2):
    return pl.pallas_call(
        _tc_b_body,
        grid=(GRID,),
        in_specs=[
            pl.BlockSpec((2, BN, 128), lambda i: (0, i, 0)),
            pl.BlockSpec((2, 128, CH), lambda i: (0, 0, 0)),
            pl.BlockSpec((2, CH), lambda i: (0, 0)),
        ],
        out_specs=[
            pl.BlockSpec((BN, W2AUG), lambda i: (i, 0)),
            pl.BlockSpec((BN, 16), lambda i: (i, 0)),
        ],
        out_shape=[
            jax.ShapeDtypeStruct((N, W2AUG), F32),
            jax.ShapeDtypeStruct((N, 16), F32),
        ],
    )(h1, W2p, a2)


def _sc2_body(hp2_hbm, al2_hbm, src_hbm, dst_hbm, accs_hbm,
              sidx, didx, rows, asrcr, adstr, wb, acc, gsem, asem, bsem):
    c = lax.axis_index("c")
    s = lax.axis_index("s")
    iota = lax.iota(I32, LANES)
    z16 = jnp.zeros((LANES,), I32)
    zf16 = jnp.zeros((LANES,), F32)

    def _zrow(i, _):
        for j in range(W2AUG // LANES):
            rows[i, pl.ds(j * LANES, LANES)] = zf16
        return 0
    lax.fori_loop(0, K, _zrow, 0)
    for r in range(7):
        pltpu.sync_copy(rows, acc.at[pl.ds(s * NT + r * K, K)])
    pltpu.sync_copy(rows.at[pl.ds(0, NT - 7 * K)],
                    acc.at[pl.ds(s * NT + 7 * K, NT - 7 * K)])
    plsc.subcore_barrier()

    ec = E // (NCORES * NSUB)
    nchunks = ec // K

    def _chunk(g, _):
        base = c * (E // NCORES) + s * ec + g * K
        pltpu.sync_copy(src_hbm.at[pl.ds(base, K)], sidx)
        pltpu.sync_copy(dst_hbm.at[pl.ds(base, K)], didx)
        d1 = pltpu.async_copy(hp2_hbm.at[sidx], rows, gsem)
        d2 = pltpu.async_copy(al2_hbm.at[sidx], asrcr, asem)
        d3 = pltpu.async_copy(al2_hbm.at[didx], adstr, bsem)
        d2.wait()
        d3.wait()
        for j in range(K // LANES):
            sl = pl.ds(j * LANES, LANES)
            l16 = iota + j * LANES
            e0 = (plsc.load_gather(asrcr, [l16, z16])
                  + plsc.load_gather(adstr, [l16, z16 + 1]))
            wb[sl] = jnp.exp(jnp.maximum(e0, 0.2 * e0))
        d1.wait()

        def _edge(e, _):
            w = wb[pl.ds(e, LANES)][0]
            for v in range(4):
                sl = pl.ds(v * LANES, LANES)
                rows[e, sl] = rows[e, sl] * w
            wsel = jnp.where(iota == 0, w, 0.0)
            sl = pl.ds(64, LANES)
            rows[e, sl] = rows[e, sl] * wsel
            return 0
        lax.fori_loop(0, K, _edge, 0)
        pltpu.sync_copy(rows, acc.at[didx], add=True)
        return 0
    lax.fori_loop(0, nchunks, _chunk, 0)
    plsc.subcore_barrier()

    hoff = c * N
    for r in range(7):
        lo = s * NT + r * K
        pltpu.sync_copy(acc.at[pl.ds(lo, K)], rows)
        pltpu.sync_copy(rows, accs_hbm.at[pl.ds(hoff + lo, K)])
    lo = s * NT + 7 * K
    sz = NT - 7 * K
    pltpu.sync_copy(acc.at[pl.ds(lo, sz)], rows.at[pl.ds(0, sz)])
    pltpu.sync_copy(rows.at[pl.ds(0, sz)], accs_hbm.at[pl.ds(hoff + lo, sz)])


def _sc2(hp2, al2, srcv, dstv):
    mesh = plsc.VectorSubcoreMesh(core_axis_name="c", subcore_axis_name="s",
                                  num_cores=NCORES, num_subcores=NSUB)
    f = functools.partial(
        pl.kernel,
        out_type=jax.ShapeDtypeStruct((2 * N, W2AUG), F32),
        mesh=mesh,
        scratch_types=[
            pltpu.VMEM((K,), I32),
            pltpu.VMEM((K,), I32),
            pltpu.VMEM((K, W2AUG), F32),
            pltpu.VMEM((K, 16), F32),
            pltpu.VMEM((K, 16), F32),
            pltpu.VMEM((K + LANES,), F32),
            pltpu.VMEM_SHARED((N, W2AUG), F32),
            pltpu.SemaphoreType.DMA,
            pltpu.SemaphoreType.DMA,
            pltpu.SemaphoreType.DMA,
        ],
        compiler_params=pltpu.CompilerParams(use_tc_tiling_on_sc=False, needs_layout_passes=False),
    )(_sc2_body)
    return f(hp2, al2, srcv, dstv)


def _tc_c_body(accs_ref, hp2_ref, al2_ref, b2_ref, out_ref):
    al = al2_ref[...]
    e = al[:, 0] + al[:, 1]
    w = jnp.exp(jnp.maximum(e, 0.2 * e))
    h = hp2_ref[...]
    num = (accs_ref[0][:, 0:CH] + accs_ref[1][:, 0:CH]
           + w[:, None] * h[:, 0:CH])
    den = accs_ref[0][:, CH] + accs_ref[1][:, CH] + w + 1e-16
    out_ref[...] = num / den[:, None] + b2_ref[...]


def _tc_c(accs, hp2, al2, b2):
    return pl.pallas_call(
        _tc_c_body,
        grid=(GRID,),
        in_specs=[
            pl.BlockSpec((2, BN, W2AUG), lambda i: (0, i, 0)),
            pl.BlockSpec((BN, W2AUG), lambda i: (i, 0)),
            pl.BlockSpec((BN, 16), lambda i: (i, 0)),
            pl.BlockSpec((1, CH), lambda i: (0, 0)),
        ],
        out_specs=pl.BlockSpec((BN, CH), lambda i: (i, 0)),
        out_shape=jax.ShapeDtypeStruct((N, CH), F32),
    )(accs, hp2, al2, b2)


def kernel(x, edge_index, W1, a_src1, a_dst1, b1, W2, a_src2, a_dst2, b2):
    ab_cols = []
    for hd in range(HEADS):
        col = jnp.zeros((HC,), F32).at[hd * CH:(hd + 1) * CH].set(a_src1[hd])
        ab_cols.append(col)
    for hd in range(HEADS):
        col = jnp.zeros((HC,), F32).at[hd * CH:(hd + 1) * CH].set(a_dst1[hd])
        ab_cols.append(col)
    Aboth = jnp.stack(ab_cols, axis=1)

    srcv = edge_index[0]
    dstv = edge_index[1]

    W2p = W2.reshape(2, 128, CH)
    a2 = jnp.concatenate([a_src2, a_dst2], axis=0)

    hp1, al1 = _tc_a(x, W1, Aboth)
    h1 = _sc1(hp1.reshape(2 * N, W1AUG), al1.reshape(2 * N, 16), srcv, dstv, b1)
    hp2, al2 = _tc_b(h1.reshape(2, N, 128), W2p, a2)
    accs = _sc2(hp2, al2, srcv, dstv)
    return _tc_c(accs.reshape(2, N, W2AUG), hp2, al2, b2.reshape(1, CH))


# speedup vs baseline: 37.6424x; 1.3266x over previous
import functools

import jax
import jax.numpy as jnp
from jax import lax
from jax.experimental import pallas as pl
from jax.experimental.pallas import tpu as pltpu
from jax.experimental.pallas import tpu_sc as plsc

F32 = jnp.float32
I32 = jnp.int32

NCORES = 2
NSUB = 16
LANES = 16

N = 10000
D_IN = 128
HEADS = 4
CH = 64
HC = HEADS * CH
E = 320000

W1AUG = 144
W2AUG = 80
K = 80
NT = N // NSUB
EPC = 64
EPC_LAST = NT - 9 * EPC

BN = 1000
GRID = N // BN


def _tc_a_body(x_ref, w1_ref, ab_ref, hp_ref, al_ref):
    h = jnp.dot(x_ref[...], w1_ref[...], preferred_element_type=F32)
    ab = jnp.dot(h, ab_ref[...], preferred_element_type=F32)
    ones2 = jnp.ones((BN, 2), F32)
    pad14 = jnp.zeros((BN, 14), F32)
    hp_ref[0] = jnp.concatenate([h[:, 0:128], ones2, pad14], axis=1)
    hp_ref[1] = jnp.concatenate([h[:, 128:256], ones2, pad14], axis=1)
    padal = jnp.zeros((BN, 12), F32)
    al_ref[0] = jnp.concatenate([ab[:, 0:2], ab[:, 4:6], padal], axis=1)
    al_ref[1] = jnp.concatenate([ab[:, 2:4], ab[:, 6:8], padal], axis=1)


def _tc_a(x, W1, Aboth):
    return pl.pallas_call(
        _tc_a_body,
        grid=(GRID,),
        in_specs=[
            pl.BlockSpec((BN, D_IN), lambda i: (i, 0)),
            pl.BlockSpec((D_IN, HC), lambda i: (0, 0)),
            pl.BlockSpec((HC, 2 * HEADS), lambda i: (0, 0)),
        ],
        out_specs=[
            pl.BlockSpec((2, BN, W1AUG), lambda i: (0, i, 0)),
            pl.BlockSpec((2, BN, 16), lambda i: (0, i, 0)),
        ],
        out_shape=[
            jax.ShapeDtypeStruct((2, N, W1AUG), F32),
            jax.ShapeDtypeStruct((2, N, 16), F32),
        ],
    )(x, W1, Aboth)


def _sc1_body(hp_hbm, al_hbm, src_hbm, dst_hbm, b1_hbm, h1_hbm,
              btab, sidx, gidx, gdidx, didx, rows, asrcr, adstr,
              sidx2, gidx2, gdidx2, didx2, rows2, asrcr2, adstr2,
              w0b, w1b, r0b, r1b, outb, acc,
              gsem, asem, bsem, gsem2, asem2, bsem2):
    c = lax.axis_index("c")
    s = lax.axis_index("s")
    iota = lax.iota(I32, LANES)
    z16 = jnp.zeros((LANES,), I32)
    zf16 = jnp.zeros((LANES,), F32)

    def _zrow(i, _):
        for j in range(W1AUG // LANES):
            rows[i, pl.ds(j * LANES, LANES)] = zf16
        return 0
    lax.fori_loop(0, EPC, _zrow, 0)
    for r in range(9):
        pltpu.sync_copy(rows.at[pl.ds(0, EPC)],
                        acc.at[pl.ds(s * NT + r * EPC, EPC)])
    pltpu.sync_copy(rows.at[pl.ds(0, EPC_LAST)],
                    acc.at[pl.ds(s * NT + 9 * EPC, EPC_LAST)])

    pltpu.sync_copy(b1_hbm.at[pl.ds(c * 128, 128)], btab)
    plsc.subcore_barrier()

    ec = E // NSUB
    npairs = ec // K // 2
    coff = c * N

    def _load(base, sidx_, didx_, gidx_, gdidx_, rows_, asrcr_, adstr_,
              sg, sa, sb):
        pltpu.sync_copy(src_hbm.at[pl.ds(base, K)], sidx_)
        pltpu.sync_copy(dst_hbm.at[pl.ds(base, K)], didx_)
        for j in range(K // LANES):
            sl = pl.ds(j * LANES, LANES)
            gidx_[sl] = sidx_[sl] + coff
            gdidx_[sl] = didx_[sl] + coff
        d1 = pltpu.async_copy(hp_hbm.at[gidx_], rows_, sg)
        d2 = pltpu.async_copy(al_hbm.at[gidx_], asrcr_, sa)
        d3 = pltpu.async_copy(al_hbm.at[gdidx_], adstr_, sb)
        return d1, d2, d3

    def _process(d1, d2, d3, rows_, asrcr_, adstr_, didx_):
        d2.wait()
        d3.wait()
        for j in range(K // LANES):
            sl = pl.ds(j * LANES, LANES)
            l16 = iota + j * LANES
            e0 = (plsc.load_gather(asrcr_, [l16, z16])
                  + plsc.load_gather(adstr_, [l16, z16 + 2]))
            e1 = (plsc.load_gather(asrcr_, [l16, z16 + 1])
                  + plsc.load_gather(adstr_, [l16, z16 + 3]))
            w0b[sl] = jnp.exp(jnp.maximum(e0, 0.2 * e0))
            w1b[sl] = jnp.exp(jnp.maximum(e1, 0.2 * e1))
        d1.wait()

        def _edge(e, _):
            w0 = w0b[pl.ds(e, LANES)][0]
            w1 = w1b[pl.ds(e, LANES)][0]
            for v in range(4):
                sl = pl.ds(v * LANES, LANES)
                rows_[e, sl] = rows_[e, sl] * w0
            for v in range(4, 8):
                sl = pl.ds(v * LANES, LANES)
                rows_[e, sl] = rows_[e, sl] * w1
            wsel = jnp.where(iota == 0, w0, w1)
            sl = pl.ds(128, LANES)
            rows_[e, sl] = rows_[e, sl] * wsel
            return 0
        lax.fori_loop(0, K, _edge, 0)
        pltpu.sync_copy(rows_, acc.at[didx_], add=True)

    def _pair(g, _):
        base = s * ec + 2 * g * K
        dA = _load(base, sidx, didx, gidx, gdidx, rows, asrcr, adstr,
                   gsem, asem, bsem)
        dB = _load(base + K, sidx2, didx2, gidx2, gdidx2, rows2, asrcr2,
                   adstr2, gsem2, asem2, bsem2)
        _process(dA[0], dA[1], dA[2], rows, asrcr, adstr, didx)
        _process(dB[0], dB[1], dB[2], rows2, asrcr2, adstr2, didx2)
        return 0
    lax.fori_loop(0, npairs, _pair, 0)
    plsc.subcore_barrier()

    for ch in range(10):
        size = EPC if ch < 9 else EPC_LAST
        lo = s * NT + ch * EPC
        pltpu.sync_copy(acc.at[pl.ds(lo, size)], rows.at[pl.ds(0, size)])
        pltpu.sync_copy(hp_hbm.at[pl.ds(coff + lo, size)],
                        rows2.at[pl.ds(0, size)])
        pltpu.sync_copy(al_hbm.at[pl.ds(coff + lo, size)],
                        asrcr.at[pl.ds(0, size)])
        for grp in range(4):
            l16 = grp * LANES + iota
            e0 = (plsc.load_gather(asrcr, [l16, z16])
                  + plsc.load_gather(asrcr, [l16, z16 + 2]))
            e1 = (plsc.load_gather(asrcr, [l16, z16 + 1])
                  + plsc.load_gather(asrcr, [l16, z16 + 3]))
            w0 = jnp.exp(jnp.maximum(e0, 0.2 * e0))
            w1 = jnp.exp(jnp.maximum(e1, 0.2 * e1))
            den0 = plsc.load_gather(rows, [l16, z16 + 128]) + w0 + 1e-16
            den1 = plsc.load_gather(rows, [l16, z16 + 129]) + w1 + 1e-16
            sl = pl.ds(grp * LANES, LANES)
            w0b[sl] = w0
            w1b[sl] = w1
            r0b[sl] = 1.0 / den0
            r1b[sl] = 1.0 / den1

        def _node(i, _):
            w0 = w0b[pl.ds(i, LANES)][0]
            w1 = w1b[pl.ds(i, LANES)][0]
            r0 = r0b[pl.ds(i, LANES)][0]
            r1 = r1b[pl.ds(i, LANES)][0]
            for v in range(8):
                sl = pl.ds(v * LANES, LANES)
                w = w0 if v < 4 else w1
                r = r0 if v < 4 else r1
                val = (rows[i, sl] + w * rows2[i, sl]) * r + btab[sl]
                outb[i, sl] = jnp.maximum(val, 0.0)
            return 0
        lax.fori_loop(0, size, _node, 0)
        pltpu.sync_copy(outb.at[pl.ds(0, size)],
                        h1_hbm.at[pl.ds(coff + lo, size)])


def _sc1(hpflat, alflat, srcv, dstv, b1p):
    mesh = plsc.VectorSubcoreMesh(core_axis_name="c", subcore_axis_name="s",
                                  num_cores=NCORES, num_subcores=NSUB)
    f = functools.partial(
        pl.kernel,
        out_type=jax.ShapeDtypeStruct((2 * N, 128), F32),
        mesh=mesh,
        scratch_types=[
            pltpu.VMEM((128,), F32),
            pltpu.VMEM((K,), I32),
            pltpu.VMEM((K,), I32),
            pltpu.VMEM((K,), I32),
            pltpu.VMEM((K,), I32),
            pltpu.VMEM((K, W1AUG), F32),
            pltpu.VMEM((K, 16), F32),
            pltpu.VMEM((K, 16), F32),
            pltpu.VMEM((K,), I32),
            pltpu.VMEM((K,), I32),
            pltpu.VMEM((K,), I32),
            pltpu.VMEM((K,), I32),
            pltpu.VMEM((K, W1AUG), F32),
            pltpu.VMEM((K, 16), F32),
            pltpu.VMEM((K, 16), F32),
            pltpu.VMEM((K + LANES,), F32),
            pltpu.VMEM((K + LANES,), F32),
            pltpu.VMEM((K + LANES,), F32),
            pltpu.VMEM((K + LANES,), F32),
            pltpu.VMEM((EPC, 128), F32),
            pltpu.VMEM_SHARED((N, W1AUG), F32),
            pltpu.SemaphoreType.DMA,
            pltpu.SemaphoreType.DMA,
            pltpu.SemaphoreType.DMA,
            pltpu.SemaphoreType.DMA,
            pltpu.SemaphoreType.DMA,
            pltpu.SemaphoreType.DMA,
        ],
        compiler_params=pltpu.CompilerParams(use_tc_tiling_on_sc=False, needs_layout_passes=False),
    )(_sc1_body)
    return f(hpflat, alflat, srcv, dstv, b1p)


def _tc_b_body(h1_ref, w2_ref, a2_ref, hp2_ref, al2_ref):
    h2 = (jnp.dot(h1_ref[0], w2_ref[0], preferred_element_type=F32)
          + jnp.dot(h1_ref[1], w2_ref[1], preferred_element_type=F32))
    al2 = jnp.dot(h2, a2_ref[...].T, preferred_element_type=F32)
    hp2_ref[...] = jnp.concatenate(
        [h2, jnp.ones((BN, 1), F32), jnp.zeros((BN, 15), F32)], axis=1)
    al2_ref[...] = jnp.concatenate([al2, jnp.zeros((BN, 14), F32)], axis=1)


def _tc_b(h1, W2p, a2):
    return pl.pallas_call(
        _tc_b_body,
        grid=(GRID,),
        in_specs=[
            pl.BlockSpec((2, BN, 128), lambda i: (0, i, 0)),
            pl.BlockSpec((2, 128, CH), lambda i: (0, 0, 0)),
            pl.BlockSpec((2, CH), lambda i: (0, 0)),
        ],
        out_specs=[
            pl.BlockSpec((BN, W2AUG), lambda i: (i, 0)),
            pl.BlockSpec((BN, 16), lambda i: (i, 0)),
        ],
        out_shape=[
            jax.ShapeDtypeStruct((N, W2AUG), F32),
            jax.ShapeDtypeStruct((N, 16), F32),
        ],
    )(h1, W2p, a2)


def _sc2_body(hp2_hbm, al2_hbm, src_hbm, dst_hbm, accs_hbm,
              sidx, didx, rows, asrcr, adstr,
              sidx2, didx2, rows2, asrcr2, adstr2,
              wb, acc, gsem, asem, bsem, gsem2, asem2, bsem2):
    c = lax.axis_index("c")
    s = lax.axis_index("s")
    iota = lax.iota(I32, LANES)
    z16 = jnp.zeros((LANES,), I32)
    zf16 = jnp.zeros((LANES,), F32)

    def _zrow(i, _):
        for j in range(W2AUG // LANES):
            rows[i, pl.ds(j * LANES, LANES)] = zf16
        return 0
    lax.fori_loop(0, K, _zrow, 0)
    for r in range(7):
        pltpu.sync_copy(rows, acc.at[pl.ds(s * NT + r * K, K)])
    pltpu.sync_copy(rows.at[pl.ds(0, NT - 7 * K)],
                    acc.at[pl.ds(s * NT + 7 * K, NT - 7 * K)])
    plsc.subcore_barrier()

    ec = E // (NCORES * NSUB)
    nchunks = ec // K
    npairs = nchunks // 2
    tbase = c * (E // NCORES) + s * ec

    def _load(base, sidx_, didx_, rows_, asrcr_, adstr_, sg, sa, sb):
        pltpu.sync_copy(src_hbm.at[pl.ds(base, K)], sidx_)
        pltpu.sync_copy(dst_hbm.at[pl.ds(base, K)], didx_)
        d1 = pltpu.async_copy(hp2_hbm.at[sidx_], rows_, sg)
        d2 = pltpu.async_copy(al2_hbm.at[sidx_], asrcr_, sa)
        d3 = pltpu.async_copy(al2_hbm.at[didx_], adstr_, sb)
        return d1, d2, d3

    def _process(d1, d2, d3, rows_, asrcr_, adstr_, didx_):
        d2.wait()
        d3.wait()
        for j in range(K // LANES):
            sl = pl.ds(j * LANES, LANES)
            l16 = iota + j * LANES
            e0 = (plsc.load_gather(asrcr_, [l16, z16])
                  + plsc.load_gather(adstr_, [l16, z16 + 1]))
            wb[sl] = jnp.exp(jnp.maximum(e0, 0.2 * e0))
        d1.wait()

        def _edge(e, _):
            w = wb[pl.ds(e, LANES)][0]
            for v in range(4):
                sl = pl.ds(v * LANES, LANES)
                rows_[e, sl] = rows_[e, sl] * w
            wsel = jnp.where(iota == 0, w, 0.0)
            sl = pl.ds(64, LANES)
            rows_[e, sl] = rows_[e, sl] * wsel
            return 0
        lax.fori_loop(0, K, _edge, 0)
        pltpu.sync_copy(rows_, acc.at[didx_], add=True)

    def _pair(g, _):
        base = tbase + 2 * g * K
        dA = _load(base, sidx, didx, rows, asrcr, adstr, gsem, asem, bsem)
        dB = _load(base + K, sidx2, didx2, rows2, asrcr2, adstr2,
                   gsem2, asem2, bsem2)
        _process(dA[0], dA[1], dA[2], rows, asrcr, adstr, didx)
        _process(dB[0], dB[1], dB[2], rows2, asrcr2, adstr2, didx2)
        return 0
    lax.fori_loop(0, npairs, _pair, 0)
    dA = _load(tbase + (nchunks - 1) * K, sidx, didx, rows, asrcr, adstr,
               gsem, asem, bsem)
    _process(dA[0], dA[1], dA[2], rows, asrcr, adstr, didx)
    plsc.subcore_barrier()

    hoff = c * N
    for r in range(7):
        lo = s * NT + r * K
        pltpu.sync_copy(acc.at[pl.ds(lo, K)], rows)
        pltpu.sync_copy(rows, accs_hbm.at[pl.ds(hoff + lo, K)])
    lo = s * NT + 7 * K
    sz = NT - 7 * K
    pltpu.sync_copy(acc.at[pl.ds(lo, sz)], rows.at[pl.ds(0, sz)])
    pltpu.sync_copy(rows.at[pl.ds(0, sz)], accs_hbm.at[pl.ds(hoff + lo, sz)])


def _sc2(hp2, al2, srcv, dstv):
    mesh = plsc.VectorSubcoreMesh(core_axis_name="c", subcore_axis_name="s",
                                  num_cores=NCORES, num_subcores=NSUB)
    f = functools.partial(
        pl.kernel,
        out_type=jax.ShapeDtypeStruct((2 * N, W2AUG), F32),
        mesh=mesh,
        scratch_types=[
            pltpu.VMEM((K,), I32),
            pltpu.VMEM((K,), I32),
            pltpu.VMEM((K, W2AUG), F32),
            pltpu.VMEM((K, 16), F32),
            pltpu.VMEM((K, 16), F32),
            pltpu.VMEM((K,), I32),
            pltpu.VMEM((K,), I32),
            pltpu.VMEM((K, W2AUG), F32),
            pltpu.VMEM((K, 16), F32),
            pltpu.VMEM((K, 16), F32),
            pltpu.VMEM((K + LANES,), F32),
            pltpu.VMEM_SHARED((N, W2AUG), F32),
            pltpu.SemaphoreType.DMA,
            pltpu.SemaphoreType.DMA,
            pltpu.SemaphoreType.DMA,
            pltpu.SemaphoreType.DMA,
            pltpu.SemaphoreType.DMA,
            pltpu.SemaphoreType.DMA,
        ],
        compiler_params=pltpu.CompilerParams(use_tc_tiling_on_sc=False, needs_layout_passes=False),
    )(_sc2_body)
    return f(hp2, al2, srcv, dstv)


def _tc_c_body(accs_ref, hp2_ref, al2_ref, b2_ref, out_ref):
    al = al2_ref[...]
    e = al[:, 0] + al[:, 1]
    w = jnp.exp(jnp.maximum(e, 0.2 * e))
    h = hp2_ref[...]
    num = (accs_ref[0][:, 0:CH] + accs_ref[1][:, 0:CH]
           + w[:, None] * h[:, 0:CH])
    den = accs_ref[0][:, CH] + accs_ref[1][:, CH] + w + 1e-16
    out_ref[...] = num / den[:, None] + b2_ref[...]


def _tc_c(accs, hp2, al2, b2):
    return pl.pallas_call(
        _tc_c_body,
        grid=(GRID,),
        in_specs=[
            pl.BlockSpec((2, BN, W2AUG), lambda i: (0, i, 0)),
            pl.BlockSpec((BN, W2AUG), lambda i: (i, 0)),
            pl.BlockSpec((BN, 16), lambda i: (i, 0)),
            pl.BlockSpec((1, CH), lambda i: (0, 0)),
        ],
        out_specs=pl.BlockSpec((BN, CH), lambda i: (i, 0)),
        out_shape=jax.ShapeDtypeStruct((N, CH), F32),
    )(accs, hp2, al2, b2)


def kernel(x, edge_index, W1, a_src1, a_dst1, b1, W2, a_src2, a_dst2, b2):
    ab_cols = []
    for hd in range(HEADS):
        col = jnp.zeros((HC,), F32).at[hd * CH:(hd + 1) * CH].set(a_src1[hd])
        ab_cols.append(col)
    for hd in range(HEADS):
        col = jnp.zeros((HC,), F32).at[hd * CH:(hd + 1) * CH].set(a_dst1[hd])
        ab_cols.append(col)
    Aboth = jnp.stack(ab_cols, axis=1)

    srcv = edge_index[0]
    dstv = edge_index[1]

    W2p = W2.reshape(2, 128, CH)
    a2 = jnp.concatenate([a_src2, a_dst2], axis=0)

    hp1, al1 = _tc_a(x, W1, Aboth)
    h1 = _sc1(hp1.reshape(2 * N, W1AUG), al1.reshape(2 * N, 16), srcv, dstv, b1)
    hp2, al2 = _tc_b(h1.reshape(2, N, 128), W2p, a2)
    accs = _sc2(hp2, al2, srcv, dstv)
    return _tc_c(accs.reshape(2, N, W2AUG), hp2, al2, b2.reshape(1, CH))
